# Initial kernel scaffold; baseline (speedup 1.0000x reference)
import jax
import jax.numpy as jnp
from jax.experimental import pallas as pl


def kernel(x, tables):
    raise NotImplementedError("write your pallas kernel here")


# speedup vs baseline: 1.1440x; 1.1440x over previous
import functools

import jax
import jax.numpy as jnp
from jax import lax
from jax.experimental import pallas as pl
from jax.experimental.pallas import tpu as pltpu
from jax.experimental.pallas import tpu_sc as plsc

_N_FIELDS = 26
_VOCAB = 100000
_EMB = 16
_NUM_CORES = 2
_NUM_SUBCORES = 16
_LANES = 16


@functools.partial(jax.jit, static_argnums=(2, 3))
def _embedding_gather(x_flat, tab_flat, n_rows, chunk):
    n_workers = _NUM_CORES * _NUM_SUBCORES
    rows_per_w = n_rows // n_workers
    n_chunks = rows_per_w // chunk
    mesh = plsc.VectorSubcoreMesh(core_axis_name="c", subcore_axis_name="s")

    def body(x_hbm, tab_hbm, out_hbm, idx_v, rows_v, sem):
        wid = lax.axis_index("s") * _NUM_CORES + lax.axis_index("c")
        wbase = wid * rows_per_w
        lanes = lax.iota(jnp.int32, _LANES)

        def do_chunk(g, carry):
            base = wbase + g * chunk
            pltpu.sync_copy(x_hbm.at[pl.ds(base, chunk)], idx_v)

            def fix(j, c):
                sl = pl.ds(j * _LANES, _LANES)
                p = lanes + (base + j * _LANES)
                idx_v[sl] = idx_v[sl] + (p % _N_FIELDS) * _VOCAB
                return c

            lax.fori_loop(0, chunk // _LANES, fix, 0)
            pltpu.async_copy(tab_hbm.at[idx_v], rows_v, sem).wait()
            pltpu.sync_copy(rows_v, out_hbm.at[pl.ds(base, chunk), :])
            return carry

        lax.fori_loop(0, n_chunks, do_chunk, 0)

    return pl.kernel(
        body,
        out_type=jax.ShapeDtypeStruct((n_rows, _EMB), jnp.float32),
        mesh=mesh,
        scratch_types=[
            pltpu.VMEM((chunk,), jnp.int32),
            pltpu.VMEM((chunk, _EMB), jnp.float32),
            pltpu.SemaphoreType.DMA,
        ],
        compiler_params=pltpu.CompilerParams(use_tc_tiling_on_sc=False),
    )(x_flat, tab_flat)


def kernel(x, tables):
    batch, n_fields = x.shape
    n_rows = batch * n_fields
    x_flat = x.reshape(n_rows)
    tab_flat = tables.reshape(n_fields * _VOCAB, _EMB)
    out = _embedding_gather(x_flat, tab_flat, n_rows, 1664)
    return out.reshape(batch, n_fields * _EMB)


# speedup vs baseline: 1.1566x; 1.0111x over previous
import functools

import jax
import jax.numpy as jnp
from jax import lax
from jax.experimental import pallas as pl
from jax.experimental.pallas import tpu as pltpu
from jax.experimental.pallas import tpu_sc as plsc

_N_FIELDS = 26
_VOCAB = 100000
_EMB = 16
_NUM_CORES = 2
_NUM_SUBCORES = 16
_LANES = 16


@functools.partial(jax.jit, static_argnums=(2, 3))
def _embedding_gather(x_flat, tab_flat, n_rows, chunk):
    n_workers = _NUM_CORES * _NUM_SUBCORES
    rows_per_w = n_rows // n_workers
    n_chunks = rows_per_w // chunk
    mesh = plsc.VectorSubcoreMesh(core_axis_name="c", subcore_axis_name="s")

    def body(x_hbm, tab_hbm, out_hbm, idx_v, rows0, rows1, gsem0, gsem1,
             wsem0, wsem1):
        wid = lax.axis_index("s") * _NUM_CORES + lax.axis_index("c")
        wbase = wid * rows_per_w
        lanes = lax.iota(jnp.int32, _LANES)
        rows_bufs = (rows0, rows1)
        gsems = (gsem0, gsem1)
        wsems = (wsem0, wsem1)

        pltpu.sync_copy(x_hbm.at[pl.ds(wbase, rows_per_w)], idx_v)

        def fix_chunk(g):
            cbase = g * chunk

            def fix(j, c):
                o = cbase + j * (4 * _LANES)
                for u in range(4):
                    sl = pl.ds(o + u * _LANES, _LANES)
                    p = lanes + (wbase + o + u * _LANES)
                    idx_v[sl] = idx_v[sl] + (p % _N_FIELDS) * _VOCAB
                return c

            lax.fori_loop(0, chunk // (4 * _LANES), fix, 0)

        def start_gather(g):
            return pltpu.async_copy(
                tab_hbm.at[idx_v.at[pl.ds(g * chunk, chunk)]],
                rows_bufs[g % 2], gsems[g % 2])

        def start_write(g):
            return pltpu.async_copy(
                rows_bufs[g % 2],
                out_hbm.at[pl.ds(wbase + g * chunk, chunk), :],
                wsems[g % 2])

        fix_chunk(0)
        gathers = {0: start_gather(0)}
        writes = {}
        for g in range(1, n_chunks):
            fix_chunk(g)
            if g >= 2:
                writes[g - 2].wait()
            gathers[g] = start_gather(g)
            gathers[g - 1].wait()
            writes[g - 1] = start_write(g - 1)
        gathers[n_chunks - 1].wait()
        writes[n_chunks - 1] = start_write(n_chunks - 1)
        writes[n_chunks - 2].wait()
        writes[n_chunks - 1].wait()

    return pl.kernel(
        body,
        out_type=jax.ShapeDtypeStruct((n_rows, _EMB), jnp.float32),
        mesh=mesh,
        scratch_types=[
            pltpu.VMEM((rows_per_w,), jnp.int32),
            pltpu.VMEM((chunk, _EMB), jnp.float32),
            pltpu.VMEM((chunk, _EMB), jnp.float32),
            pltpu.SemaphoreType.DMA,
            pltpu.SemaphoreType.DMA,
            pltpu.SemaphoreType.DMA,
            pltpu.SemaphoreType.DMA,
        ],
        compiler_params=pltpu.CompilerParams(use_tc_tiling_on_sc=False),
    )(x_flat, tab_flat)


def kernel(x, tables):
    batch, n_fields = x.shape
    n_rows = batch * n_fields
    x_flat = x.reshape(n_rows)
    tab_flat = tables.reshape(n_fields * _VOCAB, _EMB)
    out = _embedding_gather(x_flat, tab_flat, n_rows, 1664)
    return out.reshape(batch, n_fields * _EMB)
